# Initial kernel scaffold; baseline (speedup 1.0000x reference)
import jax
import jax.numpy as jnp
from jax.experimental import pallas as pl


def kernel(inputs, token_table, pos_table):
    raise NotImplementedError("write your pallas kernel here")


# speedup vs baseline: 3.7176x; 3.7176x over previous
import functools

import jax
import jax.numpy as jnp
from jax import lax
from jax.experimental import pallas as pl
from jax.experimental.pallas import tpu as pltpu
from jax.experimental.pallas import tpu_sc as plsc

MAXLEN = 200
VOCAB_SIZE = 100000
EMBED_DIM = 64
BATCH = 4096
B_FLAT = BATCH * MAXLEN

NC = 2
NS = 16
L = 16
NW = NC * NS
B_PER_W = B_FLAT // NW
SEQ_PER_CHUNK = 4
C = SEQ_PER_CHUNK * MAXLEN
N_CHUNKS = B_PER_W // C
SUB = 80
N_SUB = C // SUB
VPR = EMBED_DIM // L

_mesh = plsc.VectorSubcoreMesh(core_axis_name="c", subcore_axis_name="s")


@functools.partial(
    pl.kernel,
    mesh=_mesh,
    out_type=jax.ShapeDtypeStruct((B_FLAT, EMBED_DIM), jnp.float32),
    scratch_types=[
        pltpu.VMEM((C,), jnp.int32),
        pltpu.VMEM((C, EMBED_DIM), jnp.float32),
        pltpu.VMEM((MAXLEN, EMBED_DIM), jnp.float32),
        pltpu.SemaphoreType.DMA,
    ],
    compiler_params=pltpu.CompilerParams(use_tc_tiling_on_sc=False),
)
def _embed(idx_hbm, tok_hbm, pos_hbm, out_hbm, idx_v, rows_v, pos_v, gsem):
    wid = lax.axis_index("s") * NC + lax.axis_index("c")
    base = wid * B_PER_W

    pltpu.sync_copy(pos_hbm, pos_v)

    def chunk_body(c, carry):
        start = base + c * C
        pltpu.sync_copy(idx_hbm.at[pl.ds(start, C)], idx_v)
        copies = []
        for j in range(N_SUB):
            copies.append(
                pltpu.async_copy(
                    tok_hbm.at[idx_v.at[pl.ds(j * SUB, SUB)]],
                    rows_v.at[pl.ds(j * SUB, SUB)],
                    gsem,
                )
            )
        for cp in copies:
            cp.wait()

        def p_body(p, carry2):
            for q in range(VPR):
                sl = pl.ds(q * L, L)
                pv = pos_v[p, sl]
                for s in range(SEQ_PER_CHUNK):
                    r = s * MAXLEN + p
                    rows_v[r, sl] = rows_v[r, sl] + pv
            return carry2

        lax.fori_loop(0, MAXLEN, p_body, 0)

        pltpu.sync_copy(rows_v, out_hbm.at[pl.ds(start, C)])
        return carry

    lax.fori_loop(0, N_CHUNKS, chunk_body, 0)


def kernel(inputs, token_table, pos_table):
    flat = inputs.reshape(-1).astype(jnp.int32)
    out = _embed(flat, token_table, pos_table)
    return out.reshape(BATCH, MAXLEN, EMBED_DIM)


# speedup vs baseline: 4.2282x; 1.1374x over previous
import functools

import jax
import jax.numpy as jnp
from jax import lax
from jax.experimental import pallas as pl
from jax.experimental.pallas import tpu as pltpu
from jax.experimental.pallas import tpu_sc as plsc

MAXLEN = 200
VOCAB_SIZE = 100000
EMBED_DIM = 64
BATCH = 4096
B_FLAT = BATCH * MAXLEN

NC = 2
NS = 16
L = 16
NW = NC * NS
B_PER_W = B_FLAT // NW
SEQ_PER_CHUNK = 4
C = SEQ_PER_CHUNK * MAXLEN
N_CHUNKS = B_PER_W // C
SUB = 80
N_SUB = C // SUB
VPR = EMBED_DIM // L

_mesh = plsc.VectorSubcoreMesh(core_axis_name="c", subcore_axis_name="s")


def _fire_gathers(tok_hbm, idx_v, rows_v, sem):
    for j in range(N_SUB):
        pltpu.async_copy(
            tok_hbm.at[idx_v.at[pl.ds(j * SUB, SUB)]],
            rows_v.at[pl.ds(j * SUB, SUB)],
            sem,
        )


def _wait_gathers(tok_hbm, rows_v, sem):
    pltpu.make_async_copy(tok_hbm.at[pl.ds(0, C)], rows_v, sem).wait()


def _wait_scatter(rows_v, out_hbm, sem):
    pltpu.make_async_copy(rows_v, out_hbm.at[pl.ds(0, C)], sem).wait()


def _add_pos(rows_v, pos_v):
    def p_body(p, carry):
        for q in range(VPR):
            sl = pl.ds(q * L, L)
            pv = pos_v[p, sl]
            for s in range(SEQ_PER_CHUNK):
                r = s * MAXLEN + p
                rows_v[r, sl] = rows_v[r, sl] + pv
        return carry

    lax.fori_loop(0, MAXLEN, p_body, 0)


@functools.partial(
    pl.kernel,
    mesh=_mesh,
    out_type=jax.ShapeDtypeStruct((B_FLAT, EMBED_DIM), jnp.float32),
    scratch_types=[
        pltpu.VMEM((C,), jnp.int32),
        pltpu.VMEM((C,), jnp.int32),
        pltpu.VMEM((C, EMBED_DIM), jnp.float32),
        pltpu.VMEM((C, EMBED_DIM), jnp.float32),
        pltpu.VMEM((MAXLEN, EMBED_DIM), jnp.float32),
        pltpu.SemaphoreType.DMA,
        pltpu.SemaphoreType.DMA,
        pltpu.SemaphoreType.DMA,
        pltpu.SemaphoreType.DMA,
    ],
    compiler_params=pltpu.CompilerParams(use_tc_tiling_on_sc=False),
)
def _embed(idx_hbm, tok_hbm, pos_hbm, out_hbm,
           idx0, idx1, rows0, rows1, pos_v, g0, g1, s0, s1):
    wid = lax.axis_index("s") * NC + lax.axis_index("c")
    base = wid * B_PER_W

    idx_bufs = (idx0, idx1)
    rows_bufs = (rows0, rows1)
    gsems = (g0, g1)
    ssems = (s0, s1)

    pltpu.sync_copy(pos_hbm, pos_v)

    def stage_and_fire(c, b):
        pltpu.sync_copy(idx_hbm.at[pl.ds(base + c * C, C)], idx_bufs[b])
        _fire_gathers(tok_hbm, idx_bufs[b], rows_bufs[b], gsems[b])

    def process(c, b):
        _wait_gathers(tok_hbm, rows_bufs[b], gsems[b])
        _add_pos(rows_bufs[b], pos_v)
        pltpu.async_copy(rows_bufs[b], out_hbm.at[pl.ds(base + c * C, C)],
                         ssems[b])

    stage_and_fire(0, 0)
    stage_and_fire(1, 1)
    process(0, 0)

    def pair_body(g, carry):
        for c_off, b in ((1, 1), (2, 0)):
            c = 2 * g + c_off
            nb = 1 - b
            pltpu.sync_copy(idx_hbm.at[pl.ds(base + (c + 1) * C, C)],
                            idx_bufs[nb])
            _wait_scatter(rows_bufs[nb], out_hbm, ssems[nb])
            _fire_gathers(tok_hbm, idx_bufs[nb], rows_bufs[nb], gsems[nb])
            process(c, b)
        return carry

    lax.fori_loop(0, (N_CHUNKS - 2) // 2, pair_body, 0)

    process(N_CHUNKS - 1, (N_CHUNKS - 1) % 2)
    _wait_scatter(rows_bufs[0], out_hbm, ssems[0])
    _wait_scatter(rows_bufs[1], out_hbm, ssems[1])


def kernel(inputs, token_table, pos_table):
    flat = inputs.reshape(-1).astype(jnp.int32)
    out = _embed(flat, token_table, pos_table)
    return out.reshape(BATCH, MAXLEN, EMBED_DIM)


# speedup vs baseline: 7.4219x; 1.7553x over previous
import functools

import jax
import jax.numpy as jnp
from jax import lax
from jax.experimental import pallas as pl
from jax.experimental.pallas import tpu as pltpu
from jax.experimental.pallas import tpu_sc as plsc

MAXLEN = 200
VOCAB_SIZE = 100000
EMBED_DIM = 64
BATCH = 4096
B_FLAT = BATCH * MAXLEN

NC = 2
NS = 16
L = 16
NW = NC * NS
B_PER_W = B_FLAT // NW
SEQ_PER_CHUNK = 4
C = SEQ_PER_CHUNK * MAXLEN
N_CHUNKS = B_PER_W // C
SUB = 80
N_SUB = C // SUB
VPR = EMBED_DIM // L

_mesh = plsc.VectorSubcoreMesh(core_axis_name="c", subcore_axis_name="s")


def _fire_gathers(tok_hbm, idx_v, rows_v, sem):
    for j in range(N_SUB):
        pltpu.async_copy(
            tok_hbm.at[idx_v.at[pl.ds(j * SUB, SUB)]],
            rows_v.at[pl.ds(j * SUB, SUB)],
            sem,
        )


def _wait_gathers(tok_hbm, rows_v, sem):
    pltpu.make_async_copy(tok_hbm.at[pl.ds(0, C)], rows_v, sem).wait()


def _wait_scatter(rows_v, out_hbm, sem):
    pltpu.make_async_copy(
        rows_v, out_hbm.at[pl.ds(0, C), pl.ds(0, EMBED_DIM)], sem).wait()


def _add_pos(rows_v, pos_v):
    def p_body(p, carry):
        for q in range(VPR):
            sl = pl.ds(q * L, L)
            pv = pos_v[p, sl]
            for s in range(SEQ_PER_CHUNK):
                r = s * MAXLEN + p
                rows_v[r, sl] = rows_v[r, sl] + pv
        return carry

    lax.fori_loop(0, MAXLEN, p_body, 0)


@functools.partial(
    pl.kernel,
    mesh=_mesh,
    out_type=jax.ShapeDtypeStruct((B_FLAT, 2 * EMBED_DIM), jnp.float32),
    scratch_types=[
        pltpu.VMEM((C,), jnp.int32),
        pltpu.VMEM((C,), jnp.int32),
        pltpu.VMEM((C, EMBED_DIM), jnp.float32),
        pltpu.VMEM((C, EMBED_DIM), jnp.float32),
        pltpu.VMEM((MAXLEN, EMBED_DIM), jnp.float32),
        pltpu.SemaphoreType.DMA,
        pltpu.SemaphoreType.DMA,
        pltpu.SemaphoreType.DMA,
        pltpu.SemaphoreType.DMA,
    ],
    compiler_params=pltpu.CompilerParams(use_tc_tiling_on_sc=False),
)
def _embed(idx_hbm, tok_hbm, pos_hbm, out_hbm,
           idx0, idx1, rows0, rows1, pos_v, g0, g1, s0, s1):
    wid = lax.axis_index("s") * NC + lax.axis_index("c")
    base = wid * B_PER_W

    idx_bufs = (idx0, idx1)
    rows_bufs = (rows0, rows1)
    gsems = (g0, g1)
    ssems = (s0, s1)

    pltpu.sync_copy(pos_hbm, pos_v)

    def stage_and_fire(c, b):
        pltpu.sync_copy(idx_hbm.at[pl.ds(base + c * C, C)], idx_bufs[b])
        _fire_gathers(tok_hbm, idx_bufs[b], rows_bufs[b], gsems[b])

    def process(c, b):
        _wait_gathers(tok_hbm, rows_bufs[b], gsems[b])
        _add_pos(rows_bufs[b], pos_v)
        pltpu.async_copy(
            rows_bufs[b],
            out_hbm.at[pl.ds(base + c * C, C), pl.ds(0, EMBED_DIM)],
            ssems[b])

    stage_and_fire(0, 0)
    stage_and_fire(1, 1)
    process(0, 0)

    def pair_body(g, carry):
        for c_off, b in ((1, 1), (2, 0)):
            c = 2 * g + c_off
            nb = 1 - b
            pltpu.sync_copy(idx_hbm.at[pl.ds(base + (c + 1) * C, C)],
                            idx_bufs[nb])
            _wait_scatter(rows_bufs[nb], out_hbm, ssems[nb])
            _fire_gathers(tok_hbm, idx_bufs[nb], rows_bufs[nb], gsems[nb])
            process(c, b)
        return carry

    lax.fori_loop(0, (N_CHUNKS - 2) // 2, pair_body, 0)

    process(N_CHUNKS - 1, (N_CHUNKS - 1) % 2)
    _wait_scatter(rows_bufs[0], out_hbm, ssems[0])
    _wait_scatter(rows_bufs[1], out_hbm, ssems[1])


def kernel(inputs, token_table, pos_table):
    flat = inputs.reshape(-1).astype(jnp.int32)
    out = _embed(flat, token_table, pos_table)
    return out[:, :EMBED_DIM].reshape(BATCH, MAXLEN, EMBED_DIM)
